# baseline (speedup 1.0000x reference)
import functools

import jax
import jax.numpy as jnp
from jax.experimental import pallas as pl
from jax.experimental.pallas import tpu as pltpu

_K = 32
_WIN = 128
_TB = 256
_FC = 1024
_NEG = -1e9
_R = 8


def _inv_rms(x):
    return jax.lax.rsqrt(jnp.mean(x * x, axis=-1, keepdims=True) + 1e-6)


def _dot(a, b):
    return jax.lax.dot_general(a, b, (((1,), (0,)), ((), ())),
                               preferred_element_type=jnp.float32)


def _dotT(a, b):
    return jax.lax.dot_general(a, b, (((1,), (1,)), ((), ())),
                               preferred_element_type=jnp.float32)


def _dot0(a, b):
    return jax.lax.dot_general(a, b, (((0,), (0,)), ((), ())),
                               preferred_element_type=jnp.float32)


def _softmax(a):
    m = jnp.max(a, axis=-1, keepdims=True)
    e = jnp.exp(a - m)
    return e / jnp.sum(e, axis=-1, keepdims=True)


def _topk_into(s_col, e_ref, v_ref, k):
    S = s_col.shape[0]
    iota_col = jax.lax.broadcasted_iota(jnp.int32, (S, 1), 0).astype(jnp.float32)
    iota_row = jax.lax.broadcasted_iota(jnp.int32, (1, S), 1).astype(jnp.float32)

    def body(j, s):
        m = jnp.max(s)
        idx = jnp.min(jnp.where(s == m, iota_col, jnp.float32(S)))
        e_ref[pl.ds(j, 1), :] = (iota_row == idx).astype(jnp.float32)
        v_ref[pl.ds(j, 1), :] = jnp.reshape(m, (1, 1))
        return jnp.where(iota_col == idx, -jnp.inf, s)

    jax.lax.fori_loop(0, k, body, s_col)


def _embed_body(ids_ref, *refs):
    out_ref = refs[-1]
    for j in range(_R):
        out_ref[j, :] = refs[j][0, 0, :]


def _embed_gather(ids_flat, embed):
    T = ids_flat.shape[0]
    V, D = embed.shape
    embed3 = embed.reshape(V, 1, D)

    def imap(j, i, ids):
        return (ids[i * _R + j], 0, 0)

    return pl.pallas_call(
        _embed_body,
        grid_spec=pltpu.PrefetchScalarGridSpec(
            num_scalar_prefetch=1,
            grid=(T // _R,),
            in_specs=[pl.BlockSpec((1, 1, D), functools.partial(imap, j))
                      for j in range(_R)],
            out_specs=pl.BlockSpec((_R, D), lambda i, ids: (i, 0)),
        ),
        out_shape=jax.ShapeDtypeStruct((T, D), jnp.float32),
    )(ids_flat, *([embed3] * _R))


def _light_body(x_ref, ga_ref, wqkv_ref, wo_ref, o_ref):
    x = x_ref[...]
    dl = wo_ref.shape[0]
    xn = x * _inv_rms(x) * ga_ref[...]
    qkv = _dot(xn, wqkv_ref[...])
    q = qkv[:, :dl]
    k = qkv[:, dl:2 * dl]
    v = qkv[:, 2 * dl:]
    a = _dotT(q, k) / (dl ** 0.5)
    W = x.shape[0]
    r = jax.lax.broadcasted_iota(jnp.int32, (W, W), 0)
    c = jax.lax.broadcasted_iota(jnp.int32, (W, W), 1)
    a = _softmax(jnp.where(r >= c, a, _NEG))
    o_ref[...] = x + _dot(_dot(a, v), wo_ref[...])


def _light_attn(x, ga, wqkv, wo):
    T, D = x.shape
    return pl.pallas_call(
        _light_body,
        grid=(T // _WIN,),
        in_specs=[
            pl.BlockSpec((_WIN, D), lambda i: (i, 0)),
            pl.BlockSpec((1, D), lambda i: (0, 0)),
            pl.BlockSpec(wqkv.shape, lambda i: (0, 0)),
            pl.BlockSpec(wo.shape, lambda i: (0, 0)),
        ],
        out_specs=pl.BlockSpec((_WIN, D), lambda i: (i, 0)),
        out_shape=jax.ShapeDtypeStruct((T, D), jnp.float32),
    )(x, ga.reshape(1, D), wqkv, wo)


def _heavy_body(x_ref, ga_ref, rq_ref, rk_ref, w_ref, wo_ref,
                eq_ref, oh_ref, ek_ref, vq_ref, vk_ref):
    x = x_ref[...]
    S, D = x.shape
    inv = _inv_rms(x)
    sq = _dot(x, rq_ref[...]) * inv
    sk = _dot(x, rk_ref[...]) * inv
    _topk_into(sq, eq_ref, vq_ref, _K)
    _topk_into(sk, ek_ref, vk_ref, _K)
    Eq = eq_ref[...]
    Ek = ek_ref[...]
    ga = ga_ref[...]
    xq = _dot(Eq, x)
    xq = xq * _inv_rms(xq) * ga
    xk = _dot(Ek, x)
    xk = xk * _inv_rms(xk) * ga
    qh = _dot(xq, w_ref[:, :D])
    kh = _dot(xk, w_ref[:, D:2 * D])
    vh = _dot(xk, w_ref[:, 2 * D:]) * jax.nn.sigmoid(vk_ref[...])
    ah = _dotT(qh, kh) / (D ** 0.5)
    iota_col = jax.lax.broadcasted_iota(jnp.int32, (S, 1), 0).astype(jnp.float32)
    iota_row = jax.lax.broadcasted_iota(jnp.int32, (1, S), 1).astype(jnp.float32)
    iq = _dot(Eq, iota_col)
    ik = _dotT(iota_row, Ek)
    ah = _softmax(jnp.where(iq >= ik, ah, _NEG))
    oh_ref[...] = (_dot(_dot(ah, vh), wo_ref[...])
                   * jax.nn.sigmoid(vq_ref[...]))


def _heavy_attn(x, B, ga, rq, rk, w, wo):
    T, D = x.shape
    S = T // B
    return pl.pallas_call(
        _heavy_body,
        grid=(B,),
        in_specs=[
            pl.BlockSpec((S, D), lambda b: (b, 0)),
            pl.BlockSpec((1, D), lambda b: (0, 0)),
            pl.BlockSpec((D, 1), lambda b: (0, 0)),
            pl.BlockSpec((D, 1), lambda b: (0, 0)),
            pl.BlockSpec(w.shape, lambda b: (0, 0)),
            pl.BlockSpec(wo.shape, lambda b: (0, 0)),
        ],
        out_specs=[
            pl.BlockSpec((_K, S), lambda b: (b, 0)),
            pl.BlockSpec((_K, D), lambda b: (b, 0)),
        ],
        out_shape=[
            jax.ShapeDtypeStruct((B * _K, S), jnp.float32),
            jax.ShapeDtypeStruct((B * _K, D), jnp.float32),
        ],
        scratch_shapes=[
            pltpu.VMEM((_K, S), jnp.float32),
            pltpu.VMEM((_K, 1), jnp.float32),
            pltpu.VMEM((_K, 1), jnp.float32),
        ],
    )(x, ga.reshape(1, D), (ga * rq).reshape(D, 1), (ga * rk).reshape(D, 1),
      w, wo)


def _upd_route_body(xl_ref, eq_ref, oh_ref, gc_ref, rc_ref,
                    x1_ref, ec_ref, vc_ref, xc_ref):
    x1 = xl_ref[...] + _dot0(eq_ref[...], oh_ref[...])
    x1_ref[...] = x1
    inv = _inv_rms(x1)
    sc = _dot(x1, rc_ref[...]) * inv
    _topk_into(sc, ec_ref, vc_ref, _K)
    xc = _dot(ec_ref[...], x1)
    xc_ref[...] = xc * _inv_rms(xc) * gc_ref[...]


def _upd_route(xl, Eq, oh, B, gc, rc):
    T, D = xl.shape
    S = T // B
    return pl.pallas_call(
        _upd_route_body,
        grid=(B,),
        in_specs=[
            pl.BlockSpec((S, D), lambda b: (b, 0)),
            pl.BlockSpec((_K, S), lambda b: (b, 0)),
            pl.BlockSpec((_K, D), lambda b: (b, 0)),
            pl.BlockSpec((1, D), lambda b: (0, 0)),
            pl.BlockSpec((D, 1), lambda b: (0, 0)),
        ],
        out_specs=[
            pl.BlockSpec((S, D), lambda b: (b, 0)),
            pl.BlockSpec((_K, S), lambda b: (b, 0)),
            pl.BlockSpec((_K, 1), lambda b: (b, 0)),
            pl.BlockSpec((_K, D), lambda b: (b, 0)),
        ],
        out_shape=[
            jax.ShapeDtypeStruct((T, D), jnp.float32),
            jax.ShapeDtypeStruct((B * _K, S), jnp.float32),
            jax.ShapeDtypeStruct((B * _K, 1), jnp.float32),
            jax.ShapeDtypeStruct((B * _K, D), jnp.float32),
        ],
    )(xl, Eq, oh, gc.reshape(1, D), (gc * rc).reshape(D, 1))


def _cross_body(xc_ref, enc_ref, wq_ref, wkv_ref, oc_ref):
    xc = xc_ref[...]
    enc = enc_ref[...]
    D = xc.shape[1]
    qc = _dot(xc, wq_ref[...])
    t = _dotT(qc, wkv_ref[:, :D])
    a = _dotT(t, enc) / (D ** 0.5)
    ac = _softmax(a)
    u = _dot(ac, enc)
    oc_ref[...] = _dot(u, wkv_ref[:, D:])


def _cross_attn(xc, enc, B, wq, wkv):
    TK, D = xc.shape
    SE = enc.shape[0] // B
    return pl.pallas_call(
        _cross_body,
        grid=(B,),
        in_specs=[
            pl.BlockSpec((_K, D), lambda b: (b, 0)),
            pl.BlockSpec((SE, D), lambda b: (b, 0)),
            pl.BlockSpec(wq.shape, lambda b: (0, 0)),
            pl.BlockSpec(wkv.shape, lambda b: (0, 0)),
        ],
        out_specs=pl.BlockSpec((_K, D), lambda b: (b, 0)),
        out_shape=jax.ShapeDtypeStruct((B * _K, D), jnp.float32),
    )(xc, enc, wq, wkv)


def _upd_ff_body(x1_ref, oc_ref, ec_ref, vc_ref, woc_ref, gf_ref, rf_ref,
                 x2_ref, ef_ref, vf_ref, xf_ref):
    oc = _dot(oc_ref[...], woc_ref[...]) * jax.nn.sigmoid(vc_ref[...])
    x2 = x1_ref[...] + _dot0(ec_ref[...], oc)
    x2_ref[...] = x2
    inv = _inv_rms(x2)
    sf = _dot(x2, rf_ref[...]) * inv
    _topk_into(sf, ef_ref, vf_ref, _K)
    xf = _dot(ef_ref[...], x2)
    xf_ref[...] = xf * _inv_rms(xf) * gf_ref[...]


def _upd_ff(x1, oc, Ec, vc, B, woc, gf, rf):
    T, D = x1.shape
    S = T // B
    return pl.pallas_call(
        _upd_ff_body,
        grid=(B,),
        in_specs=[
            pl.BlockSpec((S, D), lambda b: (b, 0)),
            pl.BlockSpec((_K, D), lambda b: (b, 0)),
            pl.BlockSpec((_K, S), lambda b: (b, 0)),
            pl.BlockSpec((_K, 1), lambda b: (b, 0)),
            pl.BlockSpec(woc.shape, lambda b: (0, 0)),
            pl.BlockSpec((1, D), lambda b: (0, 0)),
            pl.BlockSpec((D, 1), lambda b: (0, 0)),
        ],
        out_specs=[
            pl.BlockSpec((S, D), lambda b: (b, 0)),
            pl.BlockSpec((_K, S), lambda b: (b, 0)),
            pl.BlockSpec((_K, 1), lambda b: (b, 0)),
            pl.BlockSpec((_K, D), lambda b: (b, 0)),
        ],
        out_shape=[
            jax.ShapeDtypeStruct((T, D), jnp.float32),
            jax.ShapeDtypeStruct((B * _K, S), jnp.float32),
            jax.ShapeDtypeStruct((B * _K, 1), jnp.float32),
            jax.ShapeDtypeStruct((B * _K, D), jnp.float32),
        ],
    )(x1, oc, Ec, vc, woc, gf.reshape(1, D), (gf * rf).reshape(D, 1))


def _hff_body(xf_ref, w1_ref, w2_ref, vf_ref, hf_ref, *, nc):
    j = pl.program_id(1)
    h = jax.nn.gelu(_dot(xf_ref[...], w1_ref[...]))
    part = _dot(h, w2_ref[...])

    @pl.when(j == 0)
    def _():
        hf_ref[...] = part

    @pl.when(j > 0)
    def _():
        hf_ref[...] = hf_ref[...] + part

    @pl.when(j == nc - 1)
    def _():
        hf_ref[...] = hf_ref[...] * jax.nn.sigmoid(vf_ref[...])


def _heavy_ff(xf, vf, B, w1, w2):
    TK, D = xf.shape
    FH = w1.shape[1]
    fc = min(_FC, FH)
    nc = FH // fc
    return pl.pallas_call(
        functools.partial(_hff_body, nc=nc),
        grid=(B, nc),
        in_specs=[
            pl.BlockSpec((_K, D), lambda b, j: (b, 0)),
            pl.BlockSpec((D, fc), lambda b, j: (0, j)),
            pl.BlockSpec((fc, D), lambda b, j: (j, 0)),
            pl.BlockSpec((_K, 1), lambda b, j: (b, 0)),
        ],
        out_specs=pl.BlockSpec((_K, D), lambda b, j: (b, 0)),
        out_shape=jax.ShapeDtypeStruct((B * _K, D), jnp.float32),
    )(xf, w1, w2, vf)


def _final_body(x2_ref, gf_ref, w1_ref, w2_ref, ef_ref, hf_ref, o_ref):
    x2 = x2_ref[...]
    xn = x2 * _inv_rms(x2) * gf_ref[...]
    lf = _dot(jax.nn.gelu(_dot(xn, w1_ref[...])), w2_ref[...])
    o_ref[...] = x2 + lf + _dot0(ef_ref[...], hf_ref[...])


def _final_ff(x2, Ef, hf, B, gf, w1, w2):
    T, D = x2.shape
    S = T // B
    nt = S // _TB
    return pl.pallas_call(
        _final_body,
        grid=(B, nt),
        in_specs=[
            pl.BlockSpec((_TB, D), lambda b, n: (b * nt + n, 0)),
            pl.BlockSpec((1, D), lambda b, n: (0, 0)),
            pl.BlockSpec(w1.shape, lambda b, n: (0, 0)),
            pl.BlockSpec(w2.shape, lambda b, n: (0, 0)),
            pl.BlockSpec((_K, _TB), lambda b, n: (b, n)),
            pl.BlockSpec((_K, D), lambda b, n: (b, 0)),
        ],
        out_specs=pl.BlockSpec((_TB, D), lambda b, n: (b * nt + n, 0)),
        out_shape=jax.ShapeDtypeStruct((T, D), jnp.float32),
    )(x2, gf.reshape(1, D), w1, w2, Ef, hf)


def kernel(input_ids, encoder_hidden_states, embed, route_q, route_kv,
           route_c, route_ff, Wqkv_l, Wo_l, Wqkv_h, Wo_h, Wq_c, Wkv_c, Wo_c,
           W1_lf, W2_lf, W1_hf, W2_hf, g_a, g_c, g_f):
    B, S = input_ids.shape
    V, D = embed.shape
    L = route_q.shape[0]
    SE = encoder_hidden_states.shape[1]
    ids = input_ids.reshape(-1).astype(jnp.int32)
    x = _embed_gather(ids, embed)
    enc = encoder_hidden_states.reshape(B * SE, D)
    for l in range(L):
        xl = _light_attn(x, g_a[l], Wqkv_l[l], Wo_l[l])
        Eq, oh = _heavy_attn(x, B, g_a[l], route_q[l], route_kv[l],
                             Wqkv_h[l], Wo_h[l])
        x1, Ec, vc, xc = _upd_route(xl, Eq, oh, B, g_c[l], route_c[l])
        oc = _cross_attn(xc, enc, B, Wq_c[l], Wkv_c[l])
        x2, Ef, vf, xf = _upd_ff(x1, oc, Ec, vc, B, Wo_c[l], g_f[l],
                                 route_ff[l])
        hf = _heavy_ff(xf, vf, B, W1_hf[l], W2_hf[l])
        x = _final_ff(x2, Ef, hf, B, g_f[l], W1_lf[l], W2_lf[l])
    return x.reshape(B, S, D)
